# Initial kernel scaffold; baseline (speedup 1.0000x reference)
import functools

import jax
import jax.numpy as jnp
from jax import lax
from jax.experimental import pallas as pl
from jax.experimental.pallas import tpu as pltpu
from jax.experimental.pallas import tpu_sc as plsc

NC = 2
NS = 16
NW = NC * NS
LANES = 16

SEQ = 200
EMB = 32
ROWS_PER_CHUNK = 4
CHUNK = ROWS_PER_CHUNK * SEQ
GPIECE = 80
NPIECE = CHUNK // GPIECE


@functools.lru_cache(maxsize=None)
def _make(batch: int):
  n_tokens = batch * SEQ
  rows_per_w = batch // NW
  n_chunks = rows_per_w // ROWS_PER_CHUNK

  mesh = plsc.VectorSubcoreMesh(core_axis_name="c", subcore_axis_name="s")

  @functools.partial(
      pl.kernel,
      out_type=jax.ShapeDtypeStruct((n_tokens, EMB), jnp.float32),
      mesh=mesh,
      scratch_types=[
          pltpu.VMEM((CHUNK,), jnp.int32),
          pltpu.VMEM((CHUNK, EMB), jnp.float32),
          pltpu.VMEM((SEQ * EMB,), jnp.float32),
          pltpu.SemaphoreType.DMA,
      ],
  )
  def body(idx_hbm, table_hbm, pos_hbm, out_hbm, idx_v, rows_v, pos_v, sem):
    wid = lax.axis_index("s") * NC + lax.axis_index("c")
    pltpu.sync_copy(pos_hbm, pos_v)
    base = wid * rows_per_w * SEQ

    def chunk_body(c, carry):
      tok0 = base + c * CHUNK
      pltpu.sync_copy(idx_hbm.at[pl.ds(tok0, CHUNK)], idx_v)
      cps = [
          pltpu.async_copy(
              table_hbm.at[idx_v.at[pl.ds(g * GPIECE, GPIECE)]],
              rows_v.at[pl.ds(g * GPIECE, GPIECE)],
              sem,
          )
          for g in range(NPIECE)
      ]
      for cp in cps:
        cp.wait()

      def s_body(s, carry2):
        p0 = pos_v[pl.ds(s * EMB, LANES)]
        p1 = pos_v[pl.ds(s * EMB + LANES, LANES)]
        for r in range(ROWS_PER_CHUNK):
          t = r * SEQ + s
          rows_v[t, pl.ds(0, LANES)] = rows_v[t, pl.ds(0, LANES)] + p0
          rows_v[t, pl.ds(LANES, LANES)] = rows_v[t, pl.ds(LANES, LANES)] + p1
        return carry2

      lax.fori_loop(0, SEQ, s_body, 0)
      pltpu.sync_copy(rows_v, out_hbm.at[pl.ds(tok0, CHUNK)])
      return carry

    lax.fori_loop(0, n_chunks, chunk_body, 0)

  return body


def kernel(inputs, token_table, pos_table):
  batch, seq = inputs.shape
  flat_idx = inputs.reshape(-1).astype(jnp.int32)
  out = _make(batch)(flat_idx, token_table, pos_table.reshape(-1))
  return out.reshape(batch, seq, EMB)


# speedup vs baseline: 4.4841x; 4.4841x over previous
import functools

import jax
import jax.numpy as jnp
from jax import lax
from jax.experimental import pallas as pl
from jax.experimental.pallas import tpu as pltpu
from jax.experimental.pallas import tpu_sc as plsc

NC = 2
NS = 16
NW = NC * NS
LANES = 16

SEQ = 200
EMB = 32
ROWS_PER_CHUNK = 4
CHUNK = ROWS_PER_CHUNK * SEQ
GPIECE = 80
NPIECE = CHUNK // GPIECE


@functools.lru_cache(maxsize=None)
def _make(batch: int):
  n_tokens = batch * SEQ
  rows_per_w = batch // NW
  n_chunks = rows_per_w // ROWS_PER_CHUNK

  mesh = plsc.VectorSubcoreMesh(core_axis_name="c", subcore_axis_name="s")

  @functools.partial(
      pl.kernel,
      out_type=jax.ShapeDtypeStruct((n_tokens, EMB), jnp.float32),
      mesh=mesh,
      scratch_types=[
          pltpu.VMEM((CHUNK,), jnp.int32),
          pltpu.VMEM((CHUNK, EMB), jnp.float32),
          pltpu.VMEM((SEQ * EMB,), jnp.float32),
          pltpu.SemaphoreType.DMA,
      ],
      compiler_params=pltpu.CompilerParams(use_tc_tiling_on_sc=False),
  )
  def body(idx_hbm, table_hbm, pos_hbm, out_hbm, idx_v, rows_v, pos_v, sem):
    wid = lax.axis_index("s") * NC + lax.axis_index("c")
    pltpu.sync_copy(pos_hbm, pos_v)
    base = wid * rows_per_w * SEQ

    def chunk_body(c, carry):
      tok0 = base + c * CHUNK
      pltpu.sync_copy(idx_hbm.at[pl.ds(tok0, CHUNK)], idx_v)
      cps = [
          pltpu.async_copy(
              table_hbm.at[idx_v.at[pl.ds(g * GPIECE, GPIECE)]],
              rows_v.at[pl.ds(g * GPIECE, GPIECE)],
              sem,
          )
          for g in range(NPIECE)
      ]
      for cp in cps:
        cp.wait()

      def s_body(s, carry2):
        p0 = pos_v[pl.ds(s * EMB, LANES)]
        p1 = pos_v[pl.ds(s * EMB + LANES, LANES)]
        for r in range(ROWS_PER_CHUNK):
          t = r * SEQ + s
          rows_v[t, pl.ds(0, LANES)] = rows_v[t, pl.ds(0, LANES)] + p0
          rows_v[t, pl.ds(LANES, LANES)] = rows_v[t, pl.ds(LANES, LANES)] + p1
        return carry2

      lax.fori_loop(0, SEQ, s_body, 0)
      pltpu.sync_copy(rows_v, out_hbm.at[pl.ds(tok0, CHUNK)])
      return carry

    lax.fori_loop(0, n_chunks, chunk_body, 0)

  return body


def kernel(inputs, token_table, pos_table):
  batch, seq = inputs.shape
  flat_idx = inputs.reshape(-1).astype(jnp.int32)
  out = _make(batch)(flat_idx, token_table, pos_table.reshape(-1))
  return out.reshape(batch, seq, EMB)
